# Initial kernel scaffold; baseline (speedup 1.0000x reference)
import jax
import jax.numpy as jnp
from jax.experimental import pallas as pl


def kernel(x, stage_table, char1_table, char2_table):
    raise NotImplementedError("write your pallas kernel here")


# speedup vs baseline: 2.1865x; 2.1865x over previous
import functools

import jax
import jax.numpy as jnp
from jax import lax
from jax.experimental import pallas as pl
from jax.experimental.pallas import tpu as pltpu
from jax.experimental.pallas import tpu_sc as plsc

NUM_TABLES = 3
B = 16384
D = 64
NC = 2
NS = 16
NW = NC * NS
BPW = B // NW
CHUNK = 128
NCH = BPW // CHUNK

_mesh = plsc.VectorSubcoreMesh(core_axis_name="c", subcore_axis_name="s")


@functools.partial(
    pl.kernel,
    mesh=_mesh,
    out_type=(
        jax.ShapeDtypeStruct((B, D), jnp.float32),
        jax.ShapeDtypeStruct((B, D), jnp.float32),
        jax.ShapeDtypeStruct((B, D), jnp.float32),
    ),
    scratch_types=[
        pltpu.VMEM((NUM_TABLES, NCH, CHUNK), jnp.int32),
        pltpu.VMEM((NUM_TABLES, NCH, CHUNK, D), jnp.float32),
        pltpu.SemaphoreType.DMA,
        pltpu.SemaphoreType.DMA,
    ],
    compiler_params=pltpu.CompilerParams(use_tc_tiling_on_sc=False),
)
def _embed3(xw, t0, t1, t2, o0, o1, o2, idx_v, rows_v, gsem, wsem):
    wid = lax.axis_index("s") * NC + lax.axis_index("c")
    base = wid * BPW
    pltpu.sync_copy(xw.at[wid], idx_v)
    tables = (t0, t1, t2)
    outs = (o0, o1, o2)
    gathers = []
    for t in range(NUM_TABLES):
        for c in range(NCH):
            gathers.append(
                pltpu.async_copy(tables[t].at[idx_v.at[t, c]],
                                 rows_v.at[t, c], gsem))
    writes = []
    for t in range(NUM_TABLES):
        for c in range(NCH):
            gathers[t * NCH + c].wait()
            writes.append(
                pltpu.async_copy(rows_v.at[t, c],
                                 outs[t].at[pl.ds(base + c * CHUNK, CHUNK)],
                                 wsem))
    for w in writes:
        w.wait()


def kernel(x, stage_table, char1_table, char2_table):
    xw = x.reshape(NW, NCH, CHUNK, NUM_TABLES).transpose(0, 3, 1, 2)
    return _embed3(xw, stage_table, char1_table, char2_table)


# speedup vs baseline: 2.2783x; 1.0420x over previous
import functools

import jax
import jax.numpy as jnp
from jax import lax
from jax.experimental import pallas as pl
from jax.experimental.pallas import tpu as pltpu
from jax.experimental.pallas import tpu_sc as plsc

NUM_TABLES = 3
B = 16384
D = 64
NC = 2
NS = 16
NW = NC * NS
BPW = B // NW
CHUNK = 128
NCH = BPW // CHUNK

_mesh = plsc.VectorSubcoreMesh(core_axis_name="c", subcore_axis_name="s")


@functools.partial(
    pl.kernel,
    mesh=_mesh,
    out_type=(
        jax.ShapeDtypeStruct((B, D), jnp.float32),
        jax.ShapeDtypeStruct((B, D), jnp.float32),
        jax.ShapeDtypeStruct((B, D), jnp.float32),
    ),
    scratch_types=[
        pltpu.VMEM((NUM_TABLES, BPW), jnp.int32),
        pltpu.VMEM((NUM_TABLES, BPW, D), jnp.float32),
        pltpu.SemaphoreType.DMA,
        pltpu.SemaphoreType.DMA,
    ],
    compiler_params=pltpu.CompilerParams(use_tc_tiling_on_sc=False),
)
def _embed3(xw, t0, t1, t2, o0, o1, o2, idx_v, rows_v, gsem, wsem):
    wid = lax.axis_index("s") * NC + lax.axis_index("c")
    base = wid * BPW
    pltpu.sync_copy(xw.at[wid], idx_v)
    tables = (t0, t1, t2)
    outs = (o0, o1, o2)
    gathers = []
    for t in range(NUM_TABLES):
        gathers.append(
            pltpu.async_copy(tables[t].at[idx_v.at[t]], rows_v.at[t], gsem))
    writes = []
    for t in range(NUM_TABLES):
        gathers[t].wait()
        writes.append(
            pltpu.async_copy(rows_v.at[t],
                             outs[t].at[pl.ds(base, BPW)], wsem))
    for w in writes:
        w.wait()


def kernel(x, stage_table, char1_table, char2_table):
    xw = x.reshape(NW, BPW, NUM_TABLES).transpose(0, 2, 1)
    return _embed3(xw, stage_table, char1_table, char2_table)
